# baseline (speedup 1.0000x reference)
import functools

import jax
import jax.numpy as jnp
from jax import lax
from jax.experimental import pallas as pl
from jax.experimental.pallas import tpu as pltpu
from jax.experimental.pallas import tpu_sc as plsc

_NUM_CLASSES = 1000
_CTX_LEN = 4
_PROMPT_LEN = _CTX_LEN + 1
_EMB_DIM = 64
_LANES = 16
_VPR = _EMB_DIM // _LANES

_NC = 1
_NS = 16
_NW = _NC * _NS
_CPW = 64


_TPL = 16
_NREP = _CPW // _TPL


def _body(ctx_hbm, emb_hbm, out_hbm, ctx_v, emb_v, tmpl_v, sem_ctx, sem_emb, sem_out):
    wid = lax.axis_index("s") * _NC + lax.axis_index("c")
    base = jnp.minimum(wid * _CPW, _NUM_CLASSES - _CPW)
    cp_ctx = pltpu.async_copy(ctx_hbm, ctx_v, sem_ctx)
    cp_emb = pltpu.async_copy(
        emb_hbm.at[pl.ds(base, _CPW)], emb_v.at[:, 0, :], sem_emb
    )
    cp_ctx.wait()
    regs = [
        ctx_v[j, pl.ds(k * _LANES, _LANES)]
        for j in range(_CTX_LEN)
        for k in range(_VPR)
    ]
    for c in range(_TPL):
        for j in range(_CTX_LEN):
            for k in range(_VPR):
                tmpl_v[c, j, pl.ds(k * _LANES, _LANES)] = regs[j * _VPR + k]
    cp_outs = [
        pltpu.async_copy(
            tmpl_v,
            out_hbm.at[pl.ds(base + r * _TPL, _TPL), pl.ds(0, _CTX_LEN), :],
            sem_out,
        )
        for r in range(_NREP)
    ]
    cp_emb.wait()
    cp_outs.append(
        pltpu.async_copy(
            emb_v, out_hbm.at[pl.ds(base, _CPW), pl.ds(_CTX_LEN, 1), :], sem_out
        )
    )
    for cp in cp_outs:
        cp.wait()


@functools.partial(
    pl.kernel,
    mesh=plsc.VectorSubcoreMesh(core_axis_name="c", subcore_axis_name="s", num_cores=_NC),
    out_type=jax.ShapeDtypeStruct((_NUM_CLASSES, _PROMPT_LEN, _EMB_DIM), jnp.float32),
    scratch_types=[
        pltpu.VMEM((_CTX_LEN, _EMB_DIM), jnp.float32),
        pltpu.VMEM((_CPW, 1, _EMB_DIM), jnp.float32),
        pltpu.VMEM((_TPL, _CTX_LEN, _EMB_DIM), jnp.float32),
        pltpu.SemaphoreType.DMA,
        pltpu.SemaphoreType.DMA,
        pltpu.SemaphoreType.DMA,
    ],
)
def _sc_prompt_head(ctx_hbm, emb_hbm, out_hbm, ctx_v, emb_v, tmpl_v, sem_ctx, sem_emb, sem_out):
    _body(ctx_hbm, emb_hbm, out_hbm, ctx_v, emb_v, tmpl_v, sem_ctx, sem_emb, sem_out)


@jax.jit
def kernel(context, emb_table):
    return _sc_prompt_head(context, emb_table)


# speedup vs baseline: 1.0107x; 1.0107x over previous
import functools

import jax
import jax.numpy as jnp
from jax import lax
from jax.experimental import pallas as pl
from jax.experimental.pallas import tpu as pltpu
from jax.experimental.pallas import tpu_sc as plsc

_NUM_CLASSES = 1000
_CTX_LEN = 4
_PROMPT_LEN = _CTX_LEN + 1
_EMB_DIM = 64
_LANES = 16
_VPR = _EMB_DIM // _LANES

_NC = 1
_NS = 16
_NW = _NC * _NS
_CPW = 64


_TPL = 4
_NREP = _CPW // _TPL


def _body(ctx_hbm, emb_hbm, out_hbm, ctx_v, emb_v, tmpl_v, sem_ctx, sem_emb, sem_out):
    wid = lax.axis_index("s") * _NC + lax.axis_index("c")
    base = jnp.minimum(wid * _CPW, _NUM_CLASSES - _CPW)
    cp_ctx = pltpu.async_copy(ctx_hbm, ctx_v, sem_ctx)
    cp_emb = pltpu.async_copy(
        emb_hbm.at[pl.ds(base, _CPW)], emb_v.at[:, 0, :], sem_emb
    )
    cp_ctx.wait()
    regs = [
        ctx_v[j, pl.ds(k * _LANES, _LANES)]
        for j in range(_CTX_LEN)
        for k in range(_VPR)
    ]
    for c in range(_TPL):
        for j in range(_CTX_LEN):
            for k in range(_VPR):
                tmpl_v[c, j, pl.ds(k * _LANES, _LANES)] = regs[j * _VPR + k]
    cp_outs = [
        pltpu.async_copy(
            tmpl_v,
            out_hbm.at[pl.ds(base + r * _TPL, _TPL), pl.ds(0, _CTX_LEN), :],
            sem_out,
        )
        for r in range(_NREP)
    ]
    cp_emb.wait()
    cp_outs.append(
        pltpu.async_copy(
            emb_v, out_hbm.at[pl.ds(base, _CPW), pl.ds(_CTX_LEN, 1), :], sem_out
        )
    )
    for cp in cp_outs:
        cp.wait()


@functools.partial(
    pl.kernel,
    mesh=plsc.VectorSubcoreMesh(core_axis_name="c", subcore_axis_name="s", num_cores=_NC),
    out_type=jax.ShapeDtypeStruct((_NUM_CLASSES, _PROMPT_LEN, _EMB_DIM), jnp.float32),
    scratch_types=[
        pltpu.VMEM((_CTX_LEN, _EMB_DIM), jnp.float32),
        pltpu.VMEM((_CPW, 1, _EMB_DIM), jnp.float32),
        pltpu.VMEM((_TPL, _CTX_LEN, _EMB_DIM), jnp.float32),
        pltpu.SemaphoreType.DMA,
        pltpu.SemaphoreType.DMA,
        pltpu.SemaphoreType.DMA,
    ],
)
def _sc_prompt_head(ctx_hbm, emb_hbm, out_hbm, ctx_v, emb_v, tmpl_v, sem_ctx, sem_emb, sem_out):
    _body(ctx_hbm, emb_hbm, out_hbm, ctx_v, emb_v, tmpl_v, sem_ctx, sem_emb, sem_out)


@jax.jit
def kernel(context, emb_table):
    return _sc_prompt_head(context, emb_table)
